# baseline (speedup 1.0000x reference)
import functools

import jax
import jax.numpy as jnp
import numpy as np
from jax import lax
from jax.experimental import pallas as pl
from jax.experimental.pallas import tpu as pltpu
from jax.experimental.pallas import tpu_sc as plsc

B = 4096
D = 64
NC, NS, L = 2, 16, 16
NW = NC * NS
APT = B // NW
KP = 256
KN = 16
FB = 512
BIGI = 1 << 30
MARGIN = 1.0
EPS = 1e-6


def _rotl32(x, r):
    return ((x << np.uint32(r)) | (x >> np.uint32(32 - r))).astype(np.uint32)


def _threefry2x32(k1, k2, x1, x2):
    x1 = x1.astype(np.uint32).copy()
    x2 = x2.astype(np.uint32).copy()
    ks = [np.uint32(k1), np.uint32(k2), np.uint32(0)]
    ks[2] = np.uint32(ks[0] ^ ks[1] ^ np.uint32(0x1BD11BDA))
    rot = (13, 15, 26, 6, 17, 29, 16, 24)
    x1 += ks[0]
    x2 += ks[1]
    for r in range(5):
        for rr in rot[:4] if r % 2 == 0 else rot[4:]:
            x1 += x2
            x2 = _rotl32(x2, rr)
            x2 ^= x1
        x1 += ks[(r + 1) % 3]
        x2 += ks[(r + 2) % 3] + np.uint32(r + 1)
    return x1, x2


def _np_uniform(key, shape):
    n = int(np.prod(shape))
    o1, o2 = _threefry2x32(key[0], key[1],
                           np.zeros(n, np.uint32), np.arange(n, dtype=np.uint32))
    bits = o1 ^ o2
    f = ((bits >> np.uint32(9)) | np.uint32(0x3F800000)).view(np.float32)
    return (f - np.float32(1.0)).reshape(shape)


def _build_orders():
    s1, s2 = _threefry2x32(np.uint32(0), np.uint32(42),
                           np.zeros(2, np.uint32), np.arange(2, dtype=np.uint32))
    ka, kb = (s1[0], s2[0]), (s1[1], s2[1])
    rp = _np_uniform(ka, (B, B))
    rn = _np_uniform(kb, (B, B))
    op = np.argsort(-rp, axis=1, kind="stable").astype(np.int32)
    on = np.argsort(-rn, axis=1, kind="stable").astype(np.int32)
    return (op, on,
            np.ascontiguousarray(op[:, :KP]), np.ascontiguousarray(on[:, :KN]))


_ORDP_FULL, _ORDN_FULL, _ORDP_PRE, _ORDN_PRE = _build_orders()

try:
    _ORDP_FULL = jax.device_put(_ORDP_FULL.reshape(B * B))
    _ORDN_FULL = jax.device_put(_ORDN_FULL.reshape(B * B))
    _ORDP_PRE = jax.device_put(_ORDP_PRE)
    _ORDN_PRE = jax.device_put(_ORDN_PRE)
except Exception:
    _ORDP_FULL = _ORDP_FULL.reshape(B * B)
    _ORDN_FULL = _ORDN_FULL.reshape(B * B)


def _sqrt16(x):
    xi = lax.bitcast_convert_type(x, jnp.int32)
    yi = lax.shift_right_logical(xi, 1) + jnp.int32(0x1FBD1DF5)
    y = lax.bitcast_convert_type(yi, jnp.float32)
    for _ in range(4):
        y = 0.5 * (y + x / y)
    return y


_mesh = plsc.VectorSubcoreMesh(core_axis_name="c", subcore_axis_name="s")


@functools.partial(
    pl.kernel,
    out_type=[
        jax.ShapeDtypeStruct((NW, L), jnp.float32),
        jax.ShapeDtypeStruct((NW, L), jnp.float32),
    ],
    mesh=_mesh,
    compiler_params=pltpu.CompilerParams(needs_layout_passes=False,
                                         use_tc_tiling_on_sc=False),
    scratch_types=[
        pltpu.VMEM((B,), jnp.int32),
        pltpu.VMEM((APT, KP), jnp.int32),
        pltpu.VMEM((APT, KN), jnp.int32),
        pltpu.VMEM((FB,), jnp.int32),
        pltpu.VMEM((APT,), jnp.int32),
        pltpu.VMEM((APT,), jnp.int32),
        pltpu.VMEM((APT,), jnp.int32),
        pltpu.VMEM((APT,), jnp.int32),
        pltpu.VMEM((APT, D), jnp.float32),
        pltpu.VMEM((APT, D), jnp.float32),
        pltpu.VMEM((APT, D), jnp.float32),
        pltpu.VMEM((APT,), jnp.float32),
        pltpu.VMEM((APT,), jnp.float32),
        pltpu.VMEM((L,), jnp.float32),
        pltpu.VMEM((L,), jnp.float32),
        pltpu.SemaphoreType.DMA,
    ],
)
def _triplet_sc(emb_hbm, labels_hbm, ordp_hbm, ordn_hbm, ordpf_hbm, ordnf_hbm,
                sum_out, cnt_out,
                labels_v, ordp_v, ordn_v, fb_v, pos_v, neg_v, pf_v, nf_v,
                a_v, p_v, n_v, sap_v, san_v, sum_v, cnt_v, sem):
    wid = lax.axis_index("s") * NC + lax.axis_index("c")
    base = wid * APT
    iota = lax.iota(jnp.int32, L)
    lane0 = iota == 0

    def splat_i(x):
        return jnp.full((L,), x, jnp.int32)

    def read1(ref, idx):
        return jnp.min(plsc.load_gather(ref, [splat_i(idx)]))

    def store1(ref, idx, val):
        plsc.store_scatter(ref, [splat_i(idx)],
                           jnp.full((L,), val, ref.dtype), mask=lane0)

    pltpu.sync_copy(labels_hbm, labels_v)
    pltpu.sync_copy(ordp_hbm.at[pl.ds(base, APT)], ordp_v)
    pltpu.sync_copy(ordn_hbm.at[pl.ds(base, APT)], ordn_v)
    pltpu.sync_copy(emb_hbm.at[pl.ds(base, APT)], a_v)

    def scan_body(a_loc, carry):
        i_glob = base + a_loc
        li_v = plsc.load_gather(labels_v, [splat_i(i_glob)])
        bpv = jnp.full((L,), BIGI, jnp.int32)
        for s in range(KP // L):
            idx16 = ordp_v[a_loc, pl.ds(s * L, L)]
            labs = plsc.load_gather(labels_v, [idx16])
            m = jnp.logical_and(labs == li_v, idx16 != i_glob)
            bpv = jnp.minimum(bpv, jnp.where(m, iota + (s * L), BIGI))
        bp = jnp.min(bpv)
        fnd = bp < KP
        bpc = jnp.minimum(bp, KP - 1)
        chosen = jnp.min(plsc.load_gather(ordp_v, [splat_i(a_loc), splat_i(bpc)]))
        store1(pos_v, a_loc, jnp.where(fnd, chosen, 0))
        store1(pf_v, a_loc, fnd.astype(jnp.int32))

        idxn = ordn_v[a_loc, :]
        labsn = plsc.load_gather(labels_v, [idxn])
        bpvn = jnp.where(labsn != li_v, iota, BIGI)
        bn = jnp.min(bpvn)
        fndn = bn < KN
        bnc = jnp.minimum(bn, KN - 1)
        chn = jnp.min(plsc.load_gather(ordn_v, [splat_i(a_loc), splat_i(bnc)]))
        store1(neg_v, a_loc, jnp.where(fndn, chn, 0))
        store1(nf_v, a_loc, fndn.astype(jnp.int32))
        return carry

    lax.fori_loop(0, APT, scan_body, 0)

    def _fb_scan(i_glob, li_v, ord_hbm, is_pos):
        def cond(c):
            k, fnd, _ = c
            return jnp.logical_and(fnd == 0, k < B)

        def body(c):
            k, fnd, ch = c
            off = pl.multiple_of(i_glob * B + k, FB)
            pltpu.sync_copy(ord_hbm.at[pl.ds(off, FB)], fb_v)
            bpv = jnp.full((L,), BIGI, jnp.int32)
            for s in range(FB // L):
                idx16 = fb_v[pl.ds(s * L, L)]
                labs = plsc.load_gather(labels_v, [idx16])
                if is_pos:
                    m = jnp.logical_and(labs == li_v, idx16 != i_glob)
                else:
                    m = labs != li_v
                bpv = jnp.minimum(bpv, jnp.where(m, iota + (s * L), BIGI))
            bp = jnp.min(bpv)
            f2 = bp < FB
            ch2 = read1(fb_v, jnp.minimum(bp, FB - 1))
            return (k + FB, f2.astype(jnp.int32), jnp.where(f2, ch2, ch))

        return lax.while_loop(
            cond, body, (jnp.int32(0), jnp.int32(0), jnp.int32(0)))

    def fix_body(a_loc, carry):
        i_glob = base + a_loc
        li_v = plsc.load_gather(labels_v, [splat_i(i_glob)])

        @pl.when(read1(pf_v, a_loc) == 0)
        def _():
            _, fnd, ch = _fb_scan(i_glob, li_v, ordpf_hbm, True)
            store1(pos_v, a_loc, ch)
            store1(pf_v, a_loc, fnd)

        @pl.when(read1(nf_v, a_loc) == 0)
        def _():
            _, fnd, ch = _fb_scan(i_glob, li_v, ordnf_hbm, False)
            store1(neg_v, a_loc, ch)
            store1(nf_v, a_loc, fnd)

        return carry

    lax.fori_loop(0, APT, fix_body, 0)

    pltpu.async_copy(emb_hbm.at[pos_v], p_v, sem).wait()
    pltpu.async_copy(emb_hbm.at[neg_v], n_v, sem).wait()

    def dist_body(a_loc, carry):
        accp = jnp.zeros((L,), jnp.float32)
        accn = jnp.zeros((L,), jnp.float32)
        for c in range(D // L):
            av = a_v[a_loc, pl.ds(c * L, L)]
            pv = p_v[a_loc, pl.ds(c * L, L)]
            nv = n_v[a_loc, pl.ds(c * L, L)]
            dp = av - pv + EPS
            accp = accp + dp * dp
            dn = av - nv + EPS
            accn = accn + dn * dn
        store1(sap_v, a_loc, jnp.sum(accp))
        store1(san_v, a_loc, jnp.sum(accn))
        return carry

    lax.fori_loop(0, APT, dist_body, 0)

    s_acc = jnp.zeros((L,), jnp.float32)
    c_acc = jnp.zeros((L,), jnp.float32)
    for g in range(APT // L):
        sap = sap_v[pl.ds(g * L, L)]
        san = san_v[pl.ds(g * L, L)]
        hinge = jnp.maximum(_sqrt16(sap) - _sqrt16(san) + MARGIN, 0.0)
        vf = (pf_v[pl.ds(g * L, L)] * nf_v[pl.ds(g * L, L)]).astype(jnp.float32)
        s_acc = s_acc + hinge * vf
        c_acc = c_acc + vf
    sum_v[...] = s_acc
    cnt_v[...] = c_acc
    pltpu.sync_copy(sum_v, sum_out.at[wid])
    pltpu.sync_copy(cnt_v, cnt_out.at[wid])


def kernel(embeddings, labels):
    sums, cnts = _triplet_sc(embeddings, labels, _ORDP_PRE, _ORDN_PRE,
                             _ORDP_FULL, _ORDN_FULL)
    total = jnp.sum(sums)
    nv = jnp.sum(cnts)
    return jnp.where(nv > 0, total / jnp.maximum(nv, 1.0), 0.0)


# speedup vs baseline: 12.5677x; 12.5677x over previous
import functools

import jax
import jax.numpy as jnp
import numpy as np
from jax import lax
from jax.experimental import pallas as pl
from jax.experimental.pallas import tpu as pltpu
from jax.experimental.pallas import tpu_sc as plsc

B = 4096
D = 64
NC, NS, L = 2, 16, 16
NW = NC * NS
APT = B // NW
KPW = 256
KNW = 16
GRPW = 4
NGRP = KPW // (GRPW * L)
BIGI = 1 << 30
MARGIN = 1.0
EPS = 1e-6
ROT = (13, 15, 26, 6, 17, 29, 16, 24)


def _rotl32(x, r):
    return ((x << np.uint32(r)) | (x >> np.uint32(32 - r))).astype(np.uint32)


def _np_threefry(k1, k2, x1, x2):
    x1 = x1.astype(np.uint32).copy()
    x2 = x2.astype(np.uint32).copy()
    ks = [np.uint32(k1), np.uint32(k2), np.uint32(0)]
    ks[2] = np.uint32(ks[0] ^ ks[1] ^ np.uint32(0x1BD11BDA))
    x1 += ks[0]
    x2 += ks[1]
    for r in range(5):
        for rr in ROT[:4] if r % 2 == 0 else ROT[4:]:
            x1 += x2
            x2 = _rotl32(x2, rr)
            x2 ^= x1
        x1 += ks[(r + 1) % 3]
        x2 += ks[(r + 2) % 3] + np.uint32(r + 1)
    return x1, x2


def _np_uniform(key, shape):
    n = int(np.prod(shape))
    o1, o2 = _np_threefry(key[0], key[1],
                          np.zeros(n, np.uint32), np.arange(n, dtype=np.uint32))
    bits = o1 ^ o2
    f = ((bits >> np.uint32(9)) | np.uint32(0x3F800000)).view(np.float32)
    return (f - np.float32(1.0)).reshape(shape)


def _pack16(order, nwords):
    pre = np.ascontiguousarray(order[:, :2 * nwords]).astype(np.uint16)
    return np.ascontiguousarray(pre).view(np.uint32).view(np.int32)


def _build_tables():
    s1, s2 = _np_threefry(np.uint32(0), np.uint32(42),
                          np.zeros(2, np.uint32), np.arange(2, dtype=np.uint32))
    ka, kb = (s1[0], s2[0]), (s1[1], s2[1])
    rp = _np_uniform(ka, (B, B))
    rn = _np_uniform(kb, (B, B))
    op = np.argsort(-rp, axis=1, kind="stable").astype(np.int32)
    on = np.argsort(-rn, axis=1, kind="stable").astype(np.int32)
    keys = tuple(int(np.asarray(v, np.uint32).view(np.int32))
                 for v in (ka[0], ka[1], kb[0], kb[1]))
    return _pack16(op, KPW), _pack16(on, KNW), keys


_PREP, _PREN, _KEYS = _build_tables()
KA1, KA2, KB1, KB2 = _KEYS

try:
    _PREP_DEV = jax.device_put(_PREP)
    _PREN_DEV = jax.device_put(_PREN)
except Exception:
    _PREP_DEV, _PREN_DEV = _PREP, _PREN


def _sqrt16(x):
    xi = lax.bitcast_convert_type(x, jnp.int32)
    yi = lax.shift_right_logical(xi, 1) + jnp.int32(0x1FBD1DF5)
    y = lax.bitcast_convert_type(yi, jnp.float32)
    for _ in range(4):
        y = 0.5 * (y + x / y)
    return y


def _tf_hash16(k1, k2, x2):
    ks0 = jnp.int32(k1)
    ks1 = jnp.int32(k2)
    ks2 = ks0 ^ ks1 ^ jnp.int32(0x1BD11BDA)
    ks = (ks0, ks1, ks2)
    x1 = jnp.full((L,), ks0, jnp.int32)
    x2 = x2 + ks1
    for r in range(5):
        for rr in ROT[:4] if r % 2 == 0 else ROT[4:]:
            x1 = x1 + x2
            x2 = (x2 << rr) | lax.shift_right_logical(x2, 32 - rr)
            x2 = x2 ^ x1
        x1 = x1 + ks[(r + 1) % 3]
        x2 = x2 + (ks[(r + 2) % 3] + jnp.int32(r + 1))
    return x1 ^ x2


_mesh = plsc.VectorSubcoreMesh(core_axis_name="c", subcore_axis_name="s")


@functools.partial(
    pl.kernel,
    out_type=[
        jax.ShapeDtypeStruct((NW, L), jnp.float32),
        jax.ShapeDtypeStruct((NW, L), jnp.float32),
    ],
    mesh=_mesh,
    compiler_params=pltpu.CompilerParams(needs_layout_passes=False,
                                         use_tc_tiling_on_sc=False),
    scratch_types=[
        pltpu.VMEM((B,), jnp.int32),
        pltpu.VMEM((APT, KPW), jnp.int32),
        pltpu.VMEM((APT, KNW), jnp.int32),
        pltpu.VMEM((APT,), jnp.int32),
        pltpu.VMEM((APT,), jnp.int32),
        pltpu.VMEM((APT,), jnp.int32),
        pltpu.VMEM((APT,), jnp.int32),
        pltpu.VMEM((APT, D), jnp.float32),
        pltpu.VMEM((APT, D), jnp.float32),
        pltpu.VMEM((APT, D), jnp.float32),
        pltpu.VMEM((APT,), jnp.float32),
        pltpu.VMEM((APT,), jnp.float32),
        pltpu.VMEM((L,), jnp.float32),
        pltpu.VMEM((L,), jnp.float32),
        pltpu.SemaphoreType.DMA,
    ],
)
def _triplet_sc(emb_hbm, labels_hbm, ordp_hbm, ordn_hbm,
                sum_out, cnt_out,
                labels_v, ordp_v, ordn_v, pos_v, neg_v, pf_v, nf_v,
                a_v, p_v, n_v, sap_v, san_v, sum_v, cnt_v, sem):
    wid = lax.axis_index("s") * NC + lax.axis_index("c")
    base = wid * APT
    iota = lax.iota(jnp.int32, L)
    lane0 = iota == 0

    def splat_i(x):
        return jnp.full((L,), x, jnp.int32)

    def store1(ref, idx, val):
        plsc.store_scatter(ref, [splat_i(idx)],
                           jnp.full((L,), val, ref.dtype), mask=lane0)

    def read1(ref, idx):
        return jnp.min(plsc.load_gather(ref, [splat_i(idx)]))

    pltpu.sync_copy(labels_hbm, labels_v)
    pltpu.sync_copy(ordp_hbm.at[pl.ds(base, APT)], ordp_v)
    pltpu.sync_copy(ordn_hbm.at[pl.ds(base, APT)], ordn_v)
    pltpu.sync_copy(emb_hbm.at[pl.ds(base, APT)], a_v)

    def match_pos(ent, li_v, i_glob):
        labs = plsc.load_gather(labels_v, [ent])
        return jnp.logical_and(labs == li_v, ent != i_glob)

    def match_neg(ent, li_v, i_glob):
        labs = plsc.load_gather(labels_v, [ent])
        return labs != li_v

    def extract(ref, a_loc, bp):
        wordcol = lax.shift_right_logical(bp, 1)
        cw = jnp.min(plsc.load_gather(ref, [splat_i(a_loc), splat_i(wordcol)]))
        return jnp.where((bp & 1) == 1,
                         lax.shift_right_logical(cw, 16), cw & 0xFFFF)

    def scan_body(a_loc, carry):
        i_glob = base + a_loc
        li_v = plsc.load_gather(labels_v, [splat_i(i_glob)])

        def cond(c):
            g, bp = c
            return jnp.logical_and(bp >= BIGI, g < NGRP)

        def body(c):
            g, _ = c
            gbase = g * (GRPW * L)
            bpv = jnp.full((L,), BIGI, jnp.int32)
            for w in range(GRPW):
                cw = ordp_v[a_loc, pl.ds(gbase + w * L, L)]
                colv = (gbase + w * L + iota) * 2
                lo = cw & 0xFFFF
                hi = lax.shift_right_logical(cw, 16)
                mlo = match_pos(lo, li_v, i_glob)
                mhi = match_pos(hi, li_v, i_glob)
                bpv = jnp.minimum(bpv, jnp.where(mlo, colv, BIGI))
                bpv = jnp.minimum(bpv, jnp.where(mhi, colv + 1, BIGI))
            return (g + 1, jnp.min(bpv))

        _, bp = lax.while_loop(cond, body, (jnp.int32(0), jnp.int32(BIGI)))
        fnd = bp < BIGI
        chosen = extract(ordp_v, a_loc, jnp.minimum(bp, 2 * KPW - 1))
        store1(pos_v, a_loc, jnp.where(fnd, chosen, 0))
        store1(pf_v, a_loc, fnd.astype(jnp.int32))

        bpv = jnp.full((L,), BIGI, jnp.int32)
        cw = ordn_v[a_loc, :]
        colv = iota * 2
        lo = cw & 0xFFFF
        hi = lax.shift_right_logical(cw, 16)
        bpv = jnp.minimum(bpv, jnp.where(match_neg(lo, li_v, i_glob), colv, BIGI))
        bpv = jnp.minimum(bpv, jnp.where(match_neg(hi, li_v, i_glob), colv + 1, BIGI))
        bn = jnp.min(bpv)
        fndn = bn < BIGI
        chn = extract(ordn_v, a_loc, jnp.minimum(bn, 2 * KNW - 1))
        store1(neg_v, a_loc, jnp.where(fndn, chn, 0))
        store1(nf_v, a_loc, fndn.astype(jnp.int32))
        return carry

    lax.fori_loop(0, APT, scan_body, 0)

    def _fb_argmax(i_glob, li_v, k1, k2, is_pos):
        match = match_pos if is_pos else match_neg

        def body(it, c):
            bv, bjp = c
            jbase = it * (2 * L)
            for h in range(2):
                jv = jbase + h * L + iota
                bits = _tf_hash16(k1, k2, i_glob * B + jv)
                val = lax.bitcast_convert_type(
                    lax.shift_right_logical(bits, 9) | jnp.int32(0x3F800000),
                    jnp.float32) - 1.0
                m = match(jv, li_v, i_glob)
                upd = jnp.logical_and(m, val > bv)
                bv = jnp.where(upd, val, bv)
                bjp = jnp.where(upd, jv, bjp)
            return (bv, bjp)

        bv, bjp = lax.fori_loop(
            0, B // (2 * L), body,
            (jnp.full((L,), -1.0, jnp.float32), jnp.zeros((L,), jnp.int32)))
        vmax = jnp.max(bv)
        fnd = vmax >= 0.0
        cand = jnp.where(bv == vmax, bjp, BIGI)
        return fnd.astype(jnp.int32), jnp.where(fnd, jnp.min(cand), 0)

    def fix_body(a_loc, carry):
        i_glob = base + a_loc
        li_v = plsc.load_gather(labels_v, [splat_i(i_glob)])

        @pl.when(read1(pf_v, a_loc) == 0)
        def _():
            fnd, ch = _fb_argmax(i_glob, li_v, KA1, KA2, True)
            store1(pos_v, a_loc, ch)
            store1(pf_v, a_loc, fnd)

        @pl.when(read1(nf_v, a_loc) == 0)
        def _():
            fnd, ch = _fb_argmax(i_glob, li_v, KB1, KB2, False)
            store1(neg_v, a_loc, ch)
            store1(nf_v, a_loc, fnd)

        return carry

    lax.fori_loop(0, APT, fix_body, 0)

    pltpu.async_copy(emb_hbm.at[pos_v], p_v, sem).wait()
    pltpu.async_copy(emb_hbm.at[neg_v], n_v, sem).wait()

    def dist_body(a_loc, carry):
        accp = jnp.zeros((L,), jnp.float32)
        accn = jnp.zeros((L,), jnp.float32)
        for c in range(D // L):
            av = a_v[a_loc, pl.ds(c * L, L)]
            pv = p_v[a_loc, pl.ds(c * L, L)]
            nv = n_v[a_loc, pl.ds(c * L, L)]
            dp = av - pv + EPS
            accp = accp + dp * dp
            dn = av - nv + EPS
            accn = accn + dn * dn
        store1(sap_v, a_loc, jnp.sum(accp))
        store1(san_v, a_loc, jnp.sum(accn))
        return carry

    lax.fori_loop(0, APT, dist_body, 0)

    s_acc = jnp.zeros((L,), jnp.float32)
    c_acc = jnp.zeros((L,), jnp.float32)
    for g in range(APT // L):
        sap = sap_v[pl.ds(g * L, L)]
        san = san_v[pl.ds(g * L, L)]
        hinge = jnp.maximum(_sqrt16(sap) - _sqrt16(san) + MARGIN, 0.0)
        vf = (pf_v[pl.ds(g * L, L)] * nf_v[pl.ds(g * L, L)]).astype(jnp.float32)
        s_acc = s_acc + hinge * vf
        c_acc = c_acc + vf
    sum_v[...] = s_acc
    cnt_v[...] = c_acc
    pltpu.sync_copy(sum_v, sum_out.at[wid])
    pltpu.sync_copy(cnt_v, cnt_out.at[wid])


def kernel(embeddings, labels):
    sums, cnts = _triplet_sc(embeddings, labels, _PREP_DEV, _PREN_DEV)
    total = jnp.sum(sums)
    nv = jnp.sum(cnts)
    return jnp.where(nv > 0, total / jnp.maximum(nv, 1.0), 0.0)
